# baseline (speedup 1.0000x reference)
import functools

import jax
import jax.numpy as jnp
from jax.experimental import pallas as pl
from jax.experimental.pallas import tpu as pltpu

_CHUNKS = (2048, 2048, 4096, 8192, 16384)


def _stream_kernel(chunks, x_hbm, w_ref, o_hbm, xbuf, obuf, in_sems, out_sems):
    offs = [sum(chunks[:i]) for i in range(len(chunks))]

    def in_copy(i):
        return pltpu.make_async_copy(
            x_hbm.at[pl.ds(offs[i], chunks[i]), :],
            xbuf.at[pl.ds(offs[i], chunks[i]), :],
            in_sems.at[i],
        )

    def out_copy(i):
        return pltpu.make_async_copy(
            obuf.at[pl.ds(offs[i], chunks[i]), :],
            o_hbm.at[pl.ds(offs[i], chunks[i]), :],
            out_sems.at[i],
        )

    for i in range(len(chunks)):
        in_copy(i).start()
    wb = w_ref[...].astype(jnp.bfloat16)
    for i in range(len(chunks)):
        in_copy(i).wait()
        obuf[pl.ds(offs[i], chunks[i]), :] = jax.lax.dot_general(
            xbuf[pl.ds(offs[i], chunks[i]), :].astype(jnp.bfloat16),
            wb,
            dimension_numbers=(((1,), (1,)), ((), ())),
            preferred_element_type=jnp.float32,
        )
        out_copy(i).start()
    for i in range(len(chunks)):
        out_copy(i).wait()


def kernel(x, W):
    B, T, D = x.shape
    rows = B * T
    xf = x.reshape(rows, D)
    chunks = _CHUNKS
    y = pl.pallas_call(
        functools.partial(_stream_kernel, chunks),
        in_specs=[
            pl.BlockSpec(memory_space=pl.ANY),
            pl.BlockSpec(memory_space=pltpu.MemorySpace.VMEM),
        ],
        out_specs=pl.BlockSpec(memory_space=pl.ANY),
        out_shape=jax.ShapeDtypeStruct((rows, D), x.dtype),
        scratch_shapes=[
            pltpu.VMEM((rows, D), jnp.float32),
            pltpu.VMEM((rows, D), jnp.float32),
            pltpu.SemaphoreType.DMA((len(chunks),)),
            pltpu.SemaphoreType.DMA((len(chunks),)),
        ],
    )(xf, W)
    return y.reshape(B, T, D)


# speedup vs baseline: 1.1830x; 1.1830x over previous
import jax
import jax.numpy as jnp
from jax.experimental import pallas as pl
from jax.experimental.pallas import tpu as pltpu

_BLOCK_ROWS = 16384


def _linear_kernel(x_ref, w_ref, o_ref):
    o_ref[...] = jax.lax.dot_general(
        x_ref[...].astype(jnp.bfloat16),
        w_ref[...].astype(jnp.bfloat16),
        dimension_numbers=(((1,), (1,)), ((), ())),
        preferred_element_type=jnp.float32,
    )


def kernel(x, W):
    B, T, D = x.shape
    rows = B * T
    xf = x.reshape(rows, D)
    block = min(_BLOCK_ROWS, rows)
    grid = pl.cdiv(rows, block)
    y = pl.pallas_call(
        _linear_kernel,
        grid=(grid,),
        in_specs=[
            pl.BlockSpec((block, D), lambda i: (i, 0)),
            pl.BlockSpec((D, D), lambda i: (0, 0)),
        ],
        out_specs=pl.BlockSpec((block, D), lambda i: (i, 0)),
        out_shape=jax.ShapeDtypeStruct((rows, D), x.dtype),
        compiler_params=pltpu.CompilerParams(
            dimension_semantics=("arbitrary",),
        ),
    )(xf, W)
    return y.reshape(B, T, D)
